# Initial kernel scaffold; baseline (speedup 1.0000x reference)
import functools
import jax
import jax.numpy as jnp
from jax import lax
from jax.experimental import pallas as pl
from jax.experimental.pallas import tpu as pltpu
from jax.experimental.pallas import tpu_sc as plsc

N_NODES = 10000
NC = 2
NS = 16
L = 16

NH = N_NODES // NC
ACC_ROWS = 5120
TRASH = ACC_ROWS - 1
SUB = 1024
G = 16
NEG_SLOPE = 0.2


def _tc_matmul_fn(x_ref, w_ref, a_ref, h_ref, aux_ref):
    h = jnp.dot(x_ref[...], w_ref[...], preferred_element_type=jnp.float32)
    h_ref[...] = h
    aux_ref[...] = jnp.dot(h, a_ref[...], preferred_element_type=jnp.float32)


def _tc_matmul(x, w, attmat):
    n, din = x.shape
    dout = w.shape[1]
    blk = 1000
    grid = (n // blk,)
    return pl.pallas_call(
        _tc_matmul_fn,
        grid=grid,
        in_specs=[
            pl.BlockSpec((blk, din), lambda i: (i, 0)),
            pl.BlockSpec((din, dout), lambda i: (0, 0)),
            pl.BlockSpec((dout, 128), lambda i: (0, 0)),
        ],
        out_specs=[
            pl.BlockSpec((blk, dout), lambda i: (i, 0)),
            pl.BlockSpec((blk, 128), lambda i: (i, 0)),
        ],
        out_shape=[
            jax.ShapeDtypeStruct((n, dout), jnp.float32),
            jax.ShapeDtypeStruct((n, 128), jnp.float32),
        ],
    )(x, w, attmat)


def _make_sc_gat(d: int, tot: int, n_edges: int, relu: bool):
    C = tot // NS
    NSUB = C // SUB
    NV = SUB // L
    NGRP = SUB // G
    NTAB = N_NODES // L
    RG = 8
    NRG = NH // RG
    NRG_IT = (NRG + NS - 1) // NS

    mesh = plsc.VectorSubcoreMesh(
        core_axis_name="c", subcore_axis_name="s", num_cores=NC,
        num_subcores=NS)

    @functools.partial(
        pl.kernel,
        out_type=jax.ShapeDtypeStruct((N_NODES, d), jnp.float32),
        mesh=mesh,
        scratch_types=[
            pltpu.VMEM_SHARED((ACC_ROWS, d), jnp.float32),
            pltpu.VMEM_SHARED((ACC_ROWS,), jnp.float32),
            pltpu.VMEM((N_NODES,), jnp.float32),
            pltpu.VMEM((N_NODES,), jnp.float32),
            pltpu.VMEM((d,), jnp.float32),
            pltpu.VMEM((SUB,), jnp.int32),
            pltpu.VMEM((SUB,), jnp.int32),
            pltpu.VMEM((SUB // G, G), jnp.int32),
            pltpu.VMEM((SUB // G, G), jnp.int32),
            pltpu.VMEM((SUB // G, G), jnp.float32),
            pltpu.VMEM((G, d), jnp.float32),
            pltpu.VMEM((8, d), jnp.float32),
            pltpu.VMEM((8,), jnp.float32),
            pltpu.SemaphoreType.DMA,
        ],
    )
    def sc_gat(src_h, dst_h, as_h, ad_h, feat_h, b_h, zacc_h, zden_h,
               out_h, acc, den, as_t, ad_t, b_t, srcv, dstv,
               slb, dlb, wb, rows, nbuf, ndb, sem):
        c = lax.axis_index("c")
        s = lax.axis_index("s")
        cbase = c * NH

        zr = ACC_ROWS // NS
        pltpu.sync_copy(zacc_h.at[pl.ds(s * zr, zr)], acc.at[pl.ds(s * zr, zr)])
        pltpu.sync_copy(zden_h.at[pl.ds(s * zr, zr)], den.at[pl.ds(s * zr, zr)])
        pltpu.sync_copy(as_h, as_t)
        pltpu.sync_copy(ad_h, ad_t)
        pltpu.sync_copy(b_h, b_t)

        def mbody(i, carry):
            am, dm = carry
            am = jnp.maximum(am, as_t[pl.ds(i * L, L)])
            dm = jnp.maximum(dm, ad_t[pl.ds(i * L, L)])
            return am, dm
        am0 = as_t[pl.ds(0, L)]
        dm0 = ad_t[pl.ds(0, L)]
        am, dm = lax.fori_loop(1, NTAB, mbody, (am0, dm0))
        msum = jnp.max(am) + jnp.max(dm)
        mv = jnp.full((L,), msum, jnp.float32)
        mv = jnp.where(mv >= 0.0, mv, mv * NEG_SLOPE)

        plsc.subcore_barrier()

        def sub_body(sub, _):
            base = s * C + sub * SUB
            pltpu.sync_copy(src_h.at[pl.ds(base, SUB)], srcv)
            pltpu.sync_copy(dst_h.at[pl.ds(base, SUB)], dstv)

            def vec_body(v, _):
                s16 = srcv[pl.ds(v * L, L)]
                d16 = dstv[pl.ds(v * L, L)]
                av = plsc.load_gather(as_t, [s16])
                dv = plsc.load_gather(ad_t, [d16])
                e = av + dv
                e = jnp.where(e >= 0.0, e, e * NEG_SLOPE)
                w = jnp.exp(e - mv)
                pos = base + v * L + lax.iota(jnp.int32, L)
                keep = ((pos < n_edges)
                        & (d16 >= cbase) & (d16 < cbase + NH))
                w = jnp.where(keep, w, 0.0)
                dl = jnp.where(keep, d16 - cbase, TRASH)
                slb[v] = s16
                dlb[v] = dl
                wb[v] = w
                return 0
            lax.fori_loop(0, NV, vec_body, 0)

            def grp_body(g, _):
                pltpu.async_copy(feat_h.at[slb.at[g]], rows, sem).wait()

                def row_body(j, _):
                    wj = wb[g, j]
                    for k in range(d // L):
                        sl = pl.ds(k * L, L)
                        rows[j, sl] = rows[j, sl] * wj
                    return 0
                lax.fori_loop(0, G, row_body, 0)

                pltpu.sync_copy(wb.at[g], den.at[dlb.at[g]], add=True)
                pltpu.sync_copy(rows, acc.at[dlb.at[g]], add=True)
                return 0
            lax.fori_loop(0, NGRP, grp_body, 0)
            return 0
        lax.fori_loop(0, NSUB, sub_body, 0)

        plsc.subcore_barrier()

        RGC = 8

        def norm_body(i, _):
            gi = s + i * NS

            @pl.when(gi < NRG)
            def _():
                r0 = gi * RGC
                pltpu.sync_copy(acc.at[pl.ds(r0, RGC)], nbuf)
                pltpu.sync_copy(den.at[pl.ds(r0, RGC)], ndb)

                def row_body(j, _):
                    dj = ndb[j]
                    for k in range(d // L):
                        sl = pl.ds(k * L, L)
                        val = nbuf[j, sl] / jnp.maximum(dj, 1e-30)
                        val = val + b_t[sl]
                        if relu:
                            val = jnp.maximum(val, 0.0)
                        nbuf[j, sl] = val
                    return 0
                lax.fori_loop(0, RGC, row_body, 0)
                pltpu.sync_copy(nbuf, out_h.at[pl.ds(cbase + r0, RGC)])
            return 0
        lax.fori_loop(0, NRG_IT, norm_body, 0)

    return sc_gat


def kernel(x, edge_index, W1, att_src1, att_dst1, b1,
           W2, att_src2, att_dst2, b2):
    n = N_NODES
    e_in = edge_index.shape[1]
    n_edges = e_in + n
    tot = ((n_edges + NS * SUB - 1) // (NS * SUB)) * NS * SUB

    loop = jnp.arange(n, dtype=jnp.int32)
    src = jnp.concatenate([edge_index[0], loop])
    dst = jnp.concatenate([edge_index[1], loop])
    pad = tot - n_edges
    src_p = jnp.pad(src, (0, pad))
    dst_p = jnp.pad(dst, (0, pad))

    hid = W1.shape[1]
    out_d = W2.shape[1]
    att1 = jnp.zeros((hid, 128), jnp.float32)
    att1 = att1.at[:, 0].set(att_src1).at[:, 1].set(att_dst1)
    att2 = jnp.zeros((out_d, 128), jnp.float32)
    att2 = att2.at[:, 0].set(att_src2).at[:, 1].set(att_dst2)

    zacc1 = jnp.zeros((ACC_ROWS, hid), jnp.float32)
    zacc2 = jnp.zeros((ACC_ROWS, out_d), jnp.float32)
    zden = jnp.zeros((ACC_ROWS,), jnp.float32)

    h1, aux1 = _tc_matmul(x, W1, att1)
    sc1 = _make_sc_gat(hid, tot, n_edges, relu=True)
    h = sc1(src_p, dst_p, aux1[:, 0], aux1[:, 1], h1, b1, zacc1, zden)

    h2, aux2 = _tc_matmul(h, W2, att2)
    sc2 = _make_sc_gat(out_d, tot, n_edges, relu=False)
    x2 = sc2(src_p, dst_p, aux2[:, 0], aux2[:, 1], h2, b2, zacc2, zden)

    return x2, h


# speedup vs baseline: 1.9564x; 1.9564x over previous
import functools
import jax
import jax.numpy as jnp
from jax import lax
from jax.experimental import pallas as pl
from jax.experimental.pallas import tpu as pltpu
from jax.experimental.pallas import tpu_sc as plsc

N_NODES = 10000
NC = 2
NS = 16
NW = NC * NS
L = 16

RPT = (N_NODES + NW - 1) // NW
ACC_ROWS = 314
TRASH = ACC_ROWS - 1
NPAD = N_NODES + 48
ADT = 328
SUB = 384
G = 16
NEG_SLOPE = 0.2


def _tc_matmul_fn(x_ref, w_ref, a_ref, h_ref, aux_ref, mx_ref):
    h = jnp.dot(x_ref[...], w_ref[...], preferred_element_type=jnp.float32)
    h_ref[...] = h
    aux = jnp.dot(h, a_ref[...], preferred_element_type=jnp.float32)
    aux_ref[...] = aux
    mblk = jnp.max(aux, axis=0, keepdims=True)

    @pl.when(pl.program_id(0) == 0)
    def _():
        mx_ref[...] = mblk

    @pl.when(pl.program_id(0) > 0)
    def _():
        mx_ref[...] = jnp.maximum(mx_ref[...], mblk)


def _tc_matmul(x, w, attmat):
    n, din = x.shape
    dout = w.shape[1]
    blk = 1000
    grid = (n // blk,)
    return pl.pallas_call(
        _tc_matmul_fn,
        grid=grid,
        in_specs=[
            pl.BlockSpec((blk, din), lambda i: (i, 0)),
            pl.BlockSpec((din, dout), lambda i: (0, 0)),
            pl.BlockSpec((dout, 128), lambda i: (0, 0)),
        ],
        out_specs=[
            pl.BlockSpec((blk, dout), lambda i: (i, 0)),
            pl.BlockSpec((blk, 128), lambda i: (i, 0)),
            pl.BlockSpec((1, 128), lambda i: (0, 0)),
        ],
        out_shape=[
            jax.ShapeDtypeStruct((n, dout), jnp.float32),
            jax.ShapeDtypeStruct((n, 128), jnp.float32),
            jax.ShapeDtypeStruct((1, 128), jnp.float32),
        ],
    )(x, w, attmat)


def _make_sc_gat(d: int, tot: int, relu: bool):
    NSUB = tot // SUB
    NV = SUB // L
    NTAB = N_NODES // L
    DCOL = d
    LCAP = SUB + L

    mesh = plsc.VectorSubcoreMesh(
        core_axis_name="c", subcore_axis_name="s", num_cores=NC,
        num_subcores=NS)

    @functools.partial(
        pl.kernel,
        out_type=jax.ShapeDtypeStruct((N_NODES, d), jnp.float32),
        mesh=mesh,
        compiler_params=pltpu.CompilerParams(needs_layout_passes=False),
        scratch_types=[
            pltpu.VMEM((ACC_ROWS, d), jnp.float32),
            pltpu.VMEM((ACC_ROWS * L,), jnp.float32),
            pltpu.VMEM((NPAD,), jnp.float32),
            pltpu.VMEM((ADT,), jnp.float32),
            pltpu.VMEM((128,), jnp.float32),
            pltpu.VMEM((d,), jnp.float32),
            pltpu.VMEM((SUB,), jnp.int32),
            pltpu.VMEM((SUB,), jnp.int32),
            pltpu.VMEM((LCAP,), jnp.int32),
            pltpu.VMEM((LCAP,), jnp.int32),
            pltpu.VMEM((LCAP,), jnp.float32),
            pltpu.VMEM((G, d), jnp.float32),
            pltpu.SemaphoreType.DMA,
        ],
    )
    def sc_gat(src_h, dst_h, as_h, ad_h, mx_h, feat_h, b_h,
               out_h, acc, den, as_t, ad_t, mx_t, b_t, slb, dstv,
               slist, dlist, wlist, rows, sem):
        c = lax.axis_index("c")
        s = lax.axis_index("s")
        wid = s * NC + c
        lo = wid * RPT
        hi = jnp.minimum(lo + RPT, N_NODES)
        albase = (lo // 8) * 8

        zvec = jnp.zeros((L,), jnp.float32)

        def zbody(r, _):
            for k in range(d // L):
                acc[r, pl.ds(k * L, L)] = zvec
            den[pl.ds(r * L, L)] = zvec
            return 0
        lax.fori_loop(0, ACC_ROWS, zbody, 0)

        pltpu.sync_copy(as_h, as_t)
        pltpu.sync_copy(ad_h.at[pl.ds(albase, ADT)], ad_t)
        pltpu.sync_copy(mx_h, mx_t)
        pltpu.sync_copy(b_h, b_t)

        mrow = mx_t[pl.ds(0, L)]
        msum = mrow[0] + mrow[1]
        mv = jnp.full((L,), msum, jnp.float32)
        mv = jnp.where(mv >= 0.0, mv, mv * NEG_SLOPE)

        def sub_body(sub, _):
            base = sub * SUB
            pltpu.sync_copy(src_h.at[pl.ds(base, SUB)], slb)
            pltpu.sync_copy(dst_h.at[pl.ds(base, SUB)], dstv)

            def vec_body(v, off):
                s16 = slb[pl.ds(v * L, L)]
                d16 = dstv[pl.ds(v * L, L)]
                dli = jnp.clip(d16 - albase, 0, ADT - 1)
                av = plsc.load_gather(as_t, [s16])
                dv = plsc.load_gather(ad_t, [dli])
                e = av + dv
                e = jnp.where(e >= 0.0, e, e * NEG_SLOPE)
                w = jnp.exp(e - mv)
                keep = (d16 >= lo) & (d16 < hi)
                dl = d16 - lo
                plsc.store_compressed(slist.at[pl.ds(off, L)], s16, mask=keep)
                plsc.store_compressed(dlist.at[pl.ds(off, L)], dl, mask=keep)
                plsc.store_compressed(wlist.at[pl.ds(off, L)], w, mask=keep)
                cnt = plsc.all_reduce_population_count(keep)
                return off + cnt[0]
            cnt = lax.fori_loop(0, NV, vec_body, jnp.int32(0))

            slist[pl.ds(cnt, L)] = jnp.zeros((L,), jnp.int32)
            dlist[pl.ds(cnt, L)] = jnp.full((L,), TRASH, jnp.int32)
            wlist[pl.ds(cnt, L)] = zvec

            ngrp = (cnt + (G - 1)) // G

            def grp_body(g, _):
                pltpu.async_copy(
                    feat_h.at[slist.at[pl.ds(g * G, G)]], rows, sem).wait()
                dlv = dlist[pl.ds(g * G, G)]
                wv = wlist[pl.ds(g * G, G)]
                for j in range(G):
                    dlj = dlv[j]
                    wj = wv[j]
                    for k in range(d // L):
                        sl = pl.ds(k * L, L)
                        acc[dlj, sl] = acc[dlj, sl] + rows[j, sl] * wj
                    dsl = pl.ds(dlj * L, L)
                    den[dsl] = den[dsl] + wj
                return 0
            lax.fori_loop(0, ngrp, grp_body, 0)
            return 0
        lax.fori_loop(0, NSUB, sub_body, 0)

        def norm_body(r, _):
            @pl.when(lo + r < hi)
            def _():
                dch = den[pl.ds(r * L, L)]
                dj = jnp.maximum(dch[0], 1e-30)
                for k in range(d // L):
                    sl = pl.ds(k * L, L)
                    val = acc[r, sl] / dj + b_t[sl]
                    if relu:
                        val = jnp.maximum(val, 0.0)
                    rows[0, sl] = val
                pltpu.sync_copy(rows.at[pl.ds(0, 1)],
                                out_h.at[pl.ds(lo + r, 1)])
            return 0
        lax.fori_loop(0, RPT, norm_body, 0)

    return sc_gat


def kernel(x, edge_index, W1, att_src1, att_dst1, b1,
           W2, att_src2, att_dst2, b2):
    n = N_NODES
    e_in = edge_index.shape[1]
    n_edges = e_in + n
    tot = ((n_edges + SUB - 1) // SUB) * SUB

    loop = jnp.arange(n, dtype=jnp.int32)
    src = jnp.concatenate([edge_index[0], loop])
    dst = jnp.concatenate([edge_index[1], loop])
    pad = tot - n_edges
    src_p = jnp.pad(src, (0, pad))
    dst_p = jnp.pad(dst, (0, pad), constant_values=n)

    hid = W1.shape[1]
    out_d = W2.shape[1]
    att1 = jnp.zeros((hid, 128), jnp.float32)
    att1 = att1.at[:, 0].set(att_src1).at[:, 1].set(att_dst1)
    att2 = jnp.zeros((out_d, 128), jnp.float32)
    att2 = att2.at[:, 0].set(att_src2).at[:, 1].set(att_dst2)

    h1, aux1, mx1 = _tc_matmul(x, W1, att1)
    as1 = jnp.pad(aux1[:, 0], (0, NPAD - n))
    ad1 = jnp.pad(aux1[:, 1], (0, NPAD - n))
    sc1 = _make_sc_gat(hid, tot, relu=True)
    h = sc1(src_p, dst_p, as1, ad1, mx1.reshape(128), h1, b1)

    h2, aux2, mx2 = _tc_matmul(h, W2, att2)
    as2 = jnp.pad(aux2[:, 0], (0, NPAD - n))
    ad2 = jnp.pad(aux2[:, 1], (0, NPAD - n))
    sc2 = _make_sc_gat(out_d, tot, relu=False)
    x2 = sc2(src_p, dst_p, as2, ad2, mx2.reshape(128), h2, b2)

    return x2, h


# speedup vs baseline: 3.3531x; 1.7139x over previous
import functools
import jax
import jax.numpy as jnp
from jax import lax
from jax.experimental import pallas as pl
from jax.experimental.pallas import tpu as pltpu
from jax.experimental.pallas import tpu_sc as plsc

N_NODES = 10000
NC = 2
NS = 16
NW = NC * NS
L = 16

RPT = (N_NODES + NW - 1) // NW
ACC_ROWS = 314
TRASH = ACC_ROWS - 1
NPAD = N_NODES + 48
ADT = 328
SUB = 1024
SLOT = SUB + 2 * L
G = 16
NEG_SLOPE = 0.2


def _tc_matmul_fn(x_ref, w_ref, a_ref, h_ref, aux_ref, mx_ref):
    h = jnp.dot(x_ref[...], w_ref[...], preferred_element_type=jnp.float32)
    h_ref[...] = h
    aux = jnp.dot(h, a_ref[...], preferred_element_type=jnp.float32)
    aux_ref[...] = aux
    mblk = jnp.max(aux, axis=0, keepdims=True)

    @pl.when(pl.program_id(0) == 0)
    def _():
        mx_ref[...] = mblk

    @pl.when(pl.program_id(0) > 0)
    def _():
        mx_ref[...] = jnp.maximum(mx_ref[...], mblk)


def _tc_matmul(x, w, attmat):
    n, din = x.shape
    dout = w.shape[1]
    blk = 1000
    grid = (n // blk,)
    return pl.pallas_call(
        _tc_matmul_fn,
        grid=grid,
        in_specs=[
            pl.BlockSpec((blk, din), lambda i: (i, 0)),
            pl.BlockSpec((din, dout), lambda i: (0, 0)),
            pl.BlockSpec((dout, 128), lambda i: (0, 0)),
        ],
        out_specs=[
            pl.BlockSpec((blk, dout), lambda i: (i, 0)),
            pl.BlockSpec((blk, 128), lambda i: (i, 0)),
            pl.BlockSpec((1, 128), lambda i: (0, 0)),
        ],
        out_shape=[
            jax.ShapeDtypeStruct((n, dout), jnp.float32),
            jax.ShapeDtypeStruct((n, 128), jnp.float32),
            jax.ShapeDtypeStruct((1, 128), jnp.float32),
        ],
    )(x, w, attmat)


def _mesh():
    return plsc.VectorSubcoreMesh(
        core_axis_name="c", subcore_axis_name="s", num_cores=NC,
        num_subcores=NS)


def _make_sc_partition(tot: int):
    NSUB = tot // SUB
    NV = SUB // L

    @functools.partial(
        pl.kernel,
        out_type=[
            jax.ShapeDtypeStruct((NW, NSUB, SLOT), jnp.int32),
            jax.ShapeDtypeStruct((NW, NSUB, SLOT), jnp.int32),
        ],
        mesh=_mesh(),
        compiler_params=pltpu.CompilerParams(needs_layout_passes=False),
        scratch_types=[
            pltpu.VMEM((SUB,), jnp.int32),
            pltpu.VMEM((SUB,), jnp.int32),
            pltpu.VMEM((SLOT,), jnp.int32),
            pltpu.VMEM((SLOT,), jnp.int32),
        ],
    )
    def sc_part(src_h, dst_h, esrc_h, edl_h, slb, dstv, slist, dlist):
        c = lax.axis_index("c")
        s = lax.axis_index("s")
        wid = s * NC + c
        lo = wid * RPT
        hi = jnp.minimum(lo + RPT, N_NODES)

        def sub_body(sub, _):
            base = sub * SUB
            pltpu.sync_copy(src_h.at[pl.ds(base, SUB)], slb)
            pltpu.sync_copy(dst_h.at[pl.ds(base, SUB)], dstv)

            def vec_body(v, off):
                s16 = slb[pl.ds(v * L, L)]
                d16 = dstv[pl.ds(v * L, L)]
                keep = (d16 >= lo) & (d16 < hi)
                dl = d16 - lo
                plsc.store_compressed(slist.at[pl.ds(off, L)], s16, mask=keep)
                plsc.store_compressed(dlist.at[pl.ds(off, L)], dl, mask=keep)
                cnt = plsc.all_reduce_population_count(keep)
                return off + cnt[0]
            cnt = lax.fori_loop(0, NV, vec_body, jnp.int32(0))

            slist[pl.ds(cnt, L)] = jnp.zeros((L,), jnp.int32)
            dlist[pl.ds(cnt, L)] = jnp.full((L,), TRASH, jnp.int32)
            ng = (cnt + (L - 1)) // L
            dlist[pl.ds(SLOT - L, L)] = jnp.full((L,), ng, jnp.int32)

            pltpu.sync_copy(slist, esrc_h.at[wid, sub])
            pltpu.sync_copy(dlist, edl_h.at[wid, sub])
            return 0
        lax.fori_loop(0, NSUB, sub_body, 0)

    return sc_part


def _make_sc_gat(d: int, tot: int, relu: bool):
    NSUB = tot // SUB

    @functools.partial(
        pl.kernel,
        out_type=jax.ShapeDtypeStruct((N_NODES, d), jnp.float32),
        mesh=_mesh(),
        compiler_params=pltpu.CompilerParams(needs_layout_passes=False),
        scratch_types=[
            pltpu.VMEM((ACC_ROWS, d), jnp.float32),
            pltpu.VMEM((ACC_ROWS * L,), jnp.float32),
            pltpu.VMEM((NPAD,), jnp.float32),
            pltpu.VMEM((ADT,), jnp.float32),
            pltpu.VMEM((128,), jnp.float32),
            pltpu.VMEM((d,), jnp.float32),
            pltpu.VMEM((SLOT,), jnp.int32),
            pltpu.VMEM((SLOT,), jnp.int32),
            pltpu.VMEM((G, d), jnp.float32),
            pltpu.SemaphoreType.DMA,
        ],
    )
    def sc_gat(esrc_h, edl_h, as_h, ad_h, mx_h, feat_h, b_h,
               out_h, acc, den, as_t, ad_t, mx_t, b_t,
               slist, dlist, rows, sem):
        c = lax.axis_index("c")
        s = lax.axis_index("s")
        wid = s * NC + c
        lo = wid * RPT
        hi = jnp.minimum(lo + RPT, N_NODES)
        albase = (lo // 8) * 8
        delta = lo - albase

        zvec = jnp.zeros((L,), jnp.float32)

        def zbody(r, _):
            for k in range(d // L):
                acc[r, pl.ds(k * L, L)] = zvec
            den[pl.ds(r * L, L)] = zvec
            return 0
        lax.fori_loop(0, ACC_ROWS, zbody, 0)

        pltpu.sync_copy(as_h, as_t)
        pltpu.sync_copy(ad_h.at[pl.ds(albase, ADT)], ad_t)
        pltpu.sync_copy(mx_h, mx_t)
        pltpu.sync_copy(b_h, b_t)

        mrow = mx_t[pl.ds(0, L)]
        msum = mrow[0] + mrow[1]
        mv = jnp.full((L,), msum, jnp.float32)
        mv = jnp.where(mv >= 0.0, mv, mv * NEG_SLOPE)

        def sub_body(sub, _):
            pltpu.sync_copy(esrc_h.at[wid, sub], slist)
            pltpu.sync_copy(edl_h.at[wid, sub], dlist)
            tail = dlist[pl.ds(SLOT - L, L)]
            ng = tail[0]

            def grp_body(g, _):
                s16 = slist[pl.ds(g * G, G)]
                dl16 = dlist[pl.ds(g * G, G)]
                av = plsc.load_gather(as_t, [s16])
                dv = plsc.load_gather(ad_t, [dl16 + delta])
                e = av + dv
                e = jnp.where(e >= 0.0, e, e * NEG_SLOPE)
                w = jnp.exp(e - mv)
                pltpu.async_copy(
                    feat_h.at[slist.at[pl.ds(g * G, G)]], rows, sem).wait()
                for j in range(G):
                    dlj = dl16[j]
                    wj = w[j]
                    for k in range(d // L):
                        sl = pl.ds(k * L, L)
                        acc[dlj, sl] = acc[dlj, sl] + rows[j, sl] * wj
                    dsl = pl.ds(dlj * L, L)
                    den[dsl] = den[dsl] + wj
                return 0
            lax.fori_loop(0, ng, grp_body, 0)
            return 0
        lax.fori_loop(0, NSUB, sub_body, 0)

        def norm_body(r, _):
            @pl.when(lo + r < hi)
            def _():
                dch = den[pl.ds(r * L, L)]
                dj = jnp.maximum(dch[0], 1e-30)
                for k in range(d // L):
                    sl = pl.ds(k * L, L)
                    val = acc[r, sl] / dj + b_t[sl]
                    if relu:
                        val = jnp.maximum(val, 0.0)
                    rows[0, sl] = val
                pltpu.sync_copy(rows.at[pl.ds(0, 1)],
                                out_h.at[pl.ds(lo + r, 1)])
            return 0
        lax.fori_loop(0, RPT, norm_body, 0)

    return sc_gat


def kernel(x, edge_index, W1, att_src1, att_dst1, b1,
           W2, att_src2, att_dst2, b2):
    n = N_NODES
    e_in = edge_index.shape[1]
    n_edges = e_in + n
    tot = ((n_edges + SUB - 1) // SUB) * SUB

    loop = jnp.arange(n, dtype=jnp.int32)
    src = jnp.concatenate([edge_index[0], loop])
    dst = jnp.concatenate([edge_index[1], loop])
    pad = tot - n_edges
    src_p = jnp.pad(src, (0, pad))
    dst_p = jnp.pad(dst, (0, pad), constant_values=n)

    hid = W1.shape[1]
    out_d = W2.shape[1]
    att1 = jnp.zeros((hid, 128), jnp.float32)
    att1 = att1.at[:, 0].set(att_src1).at[:, 1].set(att_dst1)
    att2 = jnp.zeros((out_d, 128), jnp.float32)
    att2 = att2.at[:, 0].set(att_src2).at[:, 1].set(att_dst2)

    esrc, edl = _make_sc_partition(tot)(src_p, dst_p)

    h1, aux1, mx1 = _tc_matmul(x, W1, att1)
    as1 = jnp.pad(aux1[:, 0], (0, NPAD - n))
    ad1 = jnp.pad(aux1[:, 1], (0, NPAD - n))
    sc1 = _make_sc_gat(hid, tot, relu=True)
    h = sc1(esrc, edl, as1, ad1, mx1.reshape(128), h1, b1)

    h2, aux2, mx2 = _tc_matmul(h, W2, att2)
    as2 = jnp.pad(aux2[:, 0], (0, NPAD - n))
    ad2 = jnp.pad(aux2[:, 1], (0, NPAD - n))
    sc2 = _make_sc_gat(out_d, tot, relu=False)
    x2 = sc2(esrc, edl, as2, ad2, mx2.reshape(128), h2, b2)

    return x2, h


# speedup vs baseline: 6.4610x; 1.9268x over previous
import functools
import jax
import jax.numpy as jnp
from jax import lax
from jax.experimental import pallas as pl
from jax.experimental.pallas import tpu as pltpu
from jax.experimental.pallas import tpu_sc as plsc

N_NODES = 10000
NC = 2
NS = 16
NW = NC * NS
L = 16

RPT = (N_NODES + NW - 1) // NW
ACC_ROWS = 314
TRASH = ACC_ROWS - 1
NPAD = N_NODES + 48
ADT = 328
SUB = 4096
SLOT = SUB + 2 * L
G = 16
NEG_SLOPE = 0.2


def _tc_matmul_fn(x_ref, w_ref, a_ref, h_ref, aux_ref, mx_ref):
    h = jnp.dot(x_ref[...], w_ref[...], preferred_element_type=jnp.float32)
    h_ref[...] = h
    aux = jnp.dot(h, a_ref[...], preferred_element_type=jnp.float32)
    aux_ref[...] = aux
    mblk = jnp.max(aux, axis=0, keepdims=True)

    @pl.when(pl.program_id(0) == 0)
    def _():
        mx_ref[...] = mblk

    @pl.when(pl.program_id(0) > 0)
    def _():
        mx_ref[...] = jnp.maximum(mx_ref[...], mblk)


def _tc_matmul(x, w, attmat):
    n, din = x.shape
    dout = w.shape[1]
    blk = 1000
    grid = (n // blk,)
    return pl.pallas_call(
        _tc_matmul_fn,
        grid=grid,
        in_specs=[
            pl.BlockSpec((blk, din), lambda i: (i, 0)),
            pl.BlockSpec((din, dout), lambda i: (0, 0)),
            pl.BlockSpec((dout, 128), lambda i: (0, 0)),
        ],
        out_specs=[
            pl.BlockSpec((blk, dout), lambda i: (i, 0)),
            pl.BlockSpec((blk, 128), lambda i: (i, 0)),
            pl.BlockSpec((1, 128), lambda i: (0, 0)),
        ],
        out_shape=[
            jax.ShapeDtypeStruct((n, dout), jnp.float32),
            jax.ShapeDtypeStruct((n, 128), jnp.float32),
            jax.ShapeDtypeStruct((1, 128), jnp.float32),
        ],
    )(x, w, attmat)


def _mesh():
    return plsc.VectorSubcoreMesh(
        core_axis_name="c", subcore_axis_name="s", num_cores=NC,
        num_subcores=NS)


def _make_sc_partition(tot: int):
    NSUB = tot // SUB
    NV = SUB // L

    @functools.partial(
        pl.kernel,
        out_type=[
            jax.ShapeDtypeStruct((NW, NSUB, SLOT), jnp.int32),
            jax.ShapeDtypeStruct((NW, NSUB, SLOT), jnp.int32),
        ],
        mesh=_mesh(),
        compiler_params=pltpu.CompilerParams(needs_layout_passes=False),
        scratch_types=[
            pltpu.VMEM((SUB,), jnp.int32),
            pltpu.VMEM((SUB,), jnp.int32),
            pltpu.VMEM((SLOT,), jnp.int32),
            pltpu.VMEM((SLOT,), jnp.int32),
        ],
    )
    def sc_part(src_h, dst_h, esrc_h, edl_h, slb, dstv, slist, dlist):
        c = lax.axis_index("c")
        s = lax.axis_index("s")
        wid = s * NC + c
        lo = wid * RPT
        hi = jnp.minimum(lo + RPT, N_NODES)

        def sub_body(sub, _):
            base = sub * SUB
            pltpu.sync_copy(src_h.at[pl.ds(base, SUB)], slb)
            pltpu.sync_copy(dst_h.at[pl.ds(base, SUB)], dstv)

            def vec_body(v, off):
                s16 = slb[pl.ds(v * L, L)]
                d16 = dstv[pl.ds(v * L, L)]
                keep = (d16 >= lo) & (d16 < hi)
                dl = d16 - lo
                plsc.store_compressed(slist.at[pl.ds(off, L)], s16, mask=keep)
                plsc.store_compressed(dlist.at[pl.ds(off, L)], dl, mask=keep)
                cnt = plsc.all_reduce_population_count(keep)
                return off + cnt[0]
            cnt = lax.fori_loop(0, NV, vec_body, jnp.int32(0))

            slist[pl.ds(cnt, L)] = jnp.zeros((L,), jnp.int32)
            dlist[pl.ds(cnt, L)] = jnp.full((L,), TRASH, jnp.int32)
            ng = (cnt + (L - 1)) // L
            dlist[pl.ds(SLOT - L, L)] = jnp.full((L,), ng, jnp.int32)

            pltpu.sync_copy(slist, esrc_h.at[wid, sub])
            pltpu.sync_copy(dlist, edl_h.at[wid, sub])
            return 0
        lax.fori_loop(0, NSUB, sub_body, 0)

    return sc_part


def _make_sc_gat(d: int, tot: int, relu: bool):
    NSUB = tot // SUB

    @functools.partial(
        pl.kernel,
        out_type=jax.ShapeDtypeStruct((N_NODES, d), jnp.float32),
        mesh=_mesh(),
        compiler_params=pltpu.CompilerParams(needs_layout_passes=False),
        scratch_types=[
            pltpu.VMEM((ACC_ROWS, d), jnp.float32),
            pltpu.VMEM((ACC_ROWS * L,), jnp.float32),
            pltpu.VMEM((NPAD,), jnp.float32),
            pltpu.VMEM((ADT,), jnp.float32),
            pltpu.VMEM((128,), jnp.float32),
            pltpu.VMEM((d,), jnp.float32),
            pltpu.VMEM((SLOT,), jnp.int32),
            pltpu.VMEM((SLOT,), jnp.int32),
            pltpu.VMEM((G, d), jnp.float32),
            pltpu.SemaphoreType.DMA,
        ],
    )
    def sc_gat(esrc_h, edl_h, as_h, ad_h, mx_h, feat_h, b_h,
               out_h, acc, den, as_t, ad_t, mx_t, b_t,
               slist, dlist, rows, sem):
        c = lax.axis_index("c")
        s = lax.axis_index("s")
        wid = s * NC + c
        lo = wid * RPT
        hi = jnp.minimum(lo + RPT, N_NODES)
        albase = (lo // 8) * 8
        delta = lo - albase

        zvec = jnp.zeros((L,), jnp.float32)

        def zbody(r, _):
            for k in range(d // L):
                acc[r, pl.ds(k * L, L)] = zvec
            den[pl.ds(r * L, L)] = zvec
            return 0
        lax.fori_loop(0, ACC_ROWS, zbody, 0)

        pltpu.sync_copy(as_h, as_t)
        pltpu.sync_copy(ad_h.at[pl.ds(albase, ADT)], ad_t)
        pltpu.sync_copy(mx_h, mx_t)
        pltpu.sync_copy(b_h, b_t)

        mrow = mx_t[pl.ds(0, L)]
        msum = mrow[0] + mrow[1]
        mv = jnp.full((L,), msum, jnp.float32)
        mv = jnp.where(mv >= 0.0, mv, mv * NEG_SLOPE)

        def sub_body(sub, _):
            pltpu.sync_copy(esrc_h.at[wid, sub], slist)
            pltpu.sync_copy(edl_h.at[wid, sub], dlist)
            tail = dlist[pl.ds(SLOT - L, L)]
            ng = tail[0]

            def grp_body(g, _):
                cp = pltpu.async_copy(
                    feat_h.at[slist.at[pl.ds(g * G, G)]], rows, sem)
                s16 = slist[pl.ds(g * G, G)]
                dl16 = dlist[pl.ds(g * G, G)]
                av = plsc.load_gather(as_t, [s16])
                dv = plsc.load_gather(ad_t, [dl16 + delta])
                e = av + dv
                e = jnp.where(e >= 0.0, e, e * NEG_SLOPE)
                w = jnp.exp(e - mv)
                cp.wait()
                for j in range(G):
                    dlj = dl16[j]
                    wj = w[j]
                    for k in range(d // L):
                        sl = pl.ds(k * L, L)
                        acc[dlj, sl] = acc[dlj, sl] + rows[j, sl] * wj
                    dsl = pl.ds(dlj * L, L)
                    den[dsl] = den[dsl] + wj
                return 0
            lax.fori_loop(0, ng, grp_body, 0)
            return 0
        lax.fori_loop(0, NSUB, sub_body, 0)

        def norm_body(r, _):
            @pl.when(lo + r < hi)
            def _():
                dch = den[pl.ds(r * L, L)]
                dj = jnp.maximum(dch[0], 1e-30)
                for k in range(d // L):
                    sl = pl.ds(k * L, L)
                    val = acc[r, sl] / dj + b_t[sl]
                    if relu:
                        val = jnp.maximum(val, 0.0)
                    rows[0, sl] = val
                pltpu.sync_copy(rows.at[pl.ds(0, 1)],
                                out_h.at[pl.ds(lo + r, 1)])
            return 0
        lax.fori_loop(0, RPT, norm_body, 0)

    return sc_gat


def kernel(x, edge_index, W1, att_src1, att_dst1, b1,
           W2, att_src2, att_dst2, b2):
    n = N_NODES
    e_in = edge_index.shape[1]
    n_edges = e_in + n
    tot = ((n_edges + SUB - 1) // SUB) * SUB

    loop = jnp.arange(n, dtype=jnp.int32)
    src = jnp.concatenate([edge_index[0], loop])
    dst = jnp.concatenate([edge_index[1], loop])
    pad = tot - n_edges
    src_p = jnp.pad(src, (0, pad))
    dst_p = jnp.pad(dst, (0, pad), constant_values=n)

    hid = W1.shape[1]
    out_d = W2.shape[1]
    att1 = jnp.zeros((hid, 128), jnp.float32)
    att1 = att1.at[:, 0].set(att_src1).at[:, 1].set(att_dst1)
    att2 = jnp.zeros((out_d, 128), jnp.float32)
    att2 = att2.at[:, 0].set(att_src2).at[:, 1].set(att_dst2)

    esrc, edl = _make_sc_partition(tot)(src_p, dst_p)

    h1, aux1, mx1 = _tc_matmul(x, W1, att1)
    as1 = jnp.pad(aux1[:, 0], (0, NPAD - n))
    ad1 = jnp.pad(aux1[:, 1], (0, NPAD - n))
    sc1 = _make_sc_gat(hid, tot, relu=True)
    h = sc1(esrc, edl, as1, ad1, mx1.reshape(128), h1, b1)

    h2, aux2, mx2 = _tc_matmul(h, W2, att2)
    as2 = jnp.pad(aux2[:, 0], (0, NPAD - n))
    ad2 = jnp.pad(aux2[:, 1], (0, NPAD - n))
    sc2 = _make_sc_gat(out_d, tot, relu=False)
    x2 = sc2(esrc, edl, as2, ad2, mx2.reshape(128), h2, b2)

    return x2, h


# speedup vs baseline: 7.2734x; 1.1257x over previous
import functools
import jax
import jax.numpy as jnp
from jax import lax
from jax.experimental import pallas as pl
from jax.experimental.pallas import tpu as pltpu
from jax.experimental.pallas import tpu_sc as plsc

N_NODES = 10000
NC = 2
NS = 16
NW = NC * NS
L = 16

RPT = (N_NODES + NW - 1) // NW
ACC_ROWS = 314
TRASH = ACC_ROWS - 1
NPAD = N_NODES + 48
ADT = 328
SUB = 4096
SLOT = SUB + 2 * L
G = 16
NEG_SLOPE = 0.2


def _tc_matmul_fn(x_ref, w_ref, a_ref, h_ref, aux_ref, mx_ref):
    h = jnp.dot(x_ref[...], w_ref[...], preferred_element_type=jnp.float32)
    h_ref[...] = h
    aux = jnp.dot(h, a_ref[...], preferred_element_type=jnp.float32)
    aux_ref[...] = aux
    mblk = jnp.max(aux, axis=0, keepdims=True)

    @pl.when(pl.program_id(0) == 0)
    def _():
        mx_ref[...] = mblk

    @pl.when(pl.program_id(0) > 0)
    def _():
        mx_ref[...] = jnp.maximum(mx_ref[...], mblk)


def _tc_matmul(x, w, attmat):
    n, din = x.shape
    dout = w.shape[1]
    blk = 1000
    grid = (n // blk,)
    return pl.pallas_call(
        _tc_matmul_fn,
        grid=grid,
        in_specs=[
            pl.BlockSpec((blk, din), lambda i: (i, 0)),
            pl.BlockSpec((din, dout), lambda i: (0, 0)),
            pl.BlockSpec((dout, 128), lambda i: (0, 0)),
        ],
        out_specs=[
            pl.BlockSpec((blk, dout), lambda i: (i, 0)),
            pl.BlockSpec((blk, 128), lambda i: (i, 0)),
            pl.BlockSpec((1, 128), lambda i: (0, 0)),
        ],
        out_shape=[
            jax.ShapeDtypeStruct((n, dout), jnp.float32),
            jax.ShapeDtypeStruct((n, 128), jnp.float32),
            jax.ShapeDtypeStruct((1, 128), jnp.float32),
        ],
    )(x, w, attmat)


def _mesh():
    return plsc.VectorSubcoreMesh(
        core_axis_name="c", subcore_axis_name="s", num_cores=NC,
        num_subcores=NS)


def _make_sc_partition(tot: int):
    NSUB = tot // SUB
    NV = SUB // L

    @functools.partial(
        pl.kernel,
        out_type=[
            jax.ShapeDtypeStruct((NW, NSUB, SLOT), jnp.int32),
            jax.ShapeDtypeStruct((NW, NSUB, SLOT), jnp.int32),
        ],
        mesh=_mesh(),
        compiler_params=pltpu.CompilerParams(needs_layout_passes=False),
        scratch_types=[
            pltpu.VMEM((SUB,), jnp.int32),
            pltpu.VMEM((SUB,), jnp.int32),
            pltpu.VMEM((SLOT,), jnp.int32),
            pltpu.VMEM((SLOT,), jnp.int32),
        ],
    )
    def sc_part(src_h, dst_h, esrc_h, edl_h, slb, dstv, slist, dlist):
        c = lax.axis_index("c")
        s = lax.axis_index("s")
        wid = s * NC + c
        lo = wid * RPT
        hi = jnp.minimum(lo + RPT, N_NODES)

        def sub_body(sub, _):
            base = sub * SUB
            pltpu.sync_copy(src_h.at[pl.ds(base, SUB)], slb)
            pltpu.sync_copy(dst_h.at[pl.ds(base, SUB)], dstv)

            def vec_body(v, off):
                s16 = slb[pl.ds(v * L, L)]
                d16 = dstv[pl.ds(v * L, L)]
                keep = (d16 >= lo) & (d16 < hi)
                dl = d16 - lo
                plsc.store_compressed(slist.at[pl.ds(off, L)], s16, mask=keep)
                plsc.store_compressed(dlist.at[pl.ds(off, L)], dl, mask=keep)
                cnt = plsc.all_reduce_population_count(keep)
                return off + cnt[0]
            cnt = lax.fori_loop(0, NV, vec_body, jnp.int32(0))

            slist[pl.ds(cnt, L)] = jnp.zeros((L,), jnp.int32)
            dlist[pl.ds(cnt, L)] = jnp.full((L,), TRASH, jnp.int32)
            ng = (cnt + (L - 1)) // L
            dlist[pl.ds(SLOT - L, L)] = jnp.full((L,), ng, jnp.int32)

            pltpu.sync_copy(slist, esrc_h.at[wid, sub])
            pltpu.sync_copy(dlist, edl_h.at[wid, sub])
            return 0
        lax.fori_loop(0, NSUB, sub_body, 0)

    return sc_part


def _make_sc_gat(d: int, tot: int, relu: bool):
    NSUB = tot // SUB

    @functools.partial(
        pl.kernel,
        out_type=jax.ShapeDtypeStruct((N_NODES, d), jnp.float32),
        mesh=_mesh(),
        compiler_params=pltpu.CompilerParams(needs_layout_passes=False),
        scratch_types=[
            pltpu.VMEM((ACC_ROWS, d), jnp.float32),
            pltpu.VMEM((ACC_ROWS * L,), jnp.float32),
            pltpu.VMEM((NPAD,), jnp.float32),
            pltpu.VMEM((ADT,), jnp.float32),
            pltpu.VMEM((128,), jnp.float32),
            pltpu.VMEM((d,), jnp.float32),
            pltpu.VMEM((SLOT,), jnp.int32),
            pltpu.VMEM((SLOT,), jnp.int32),
            pltpu.VMEM((G, d), jnp.float32),
            pltpu.SemaphoreType.DMA,
        ],
    )
    def sc_gat(esrc_h, edl_h, as_h, ad_h, mx_h, feat_h, b_h,
               out_h, acc, den, as_t, ad_t, mx_t, b_t,
               slist, dlist, rows, sem):
        c = lax.axis_index("c")
        s = lax.axis_index("s")
        wid = s * NC + c
        lo = wid * RPT
        hi = jnp.minimum(lo + RPT, N_NODES)
        albase = (lo // 8) * 8
        delta = lo - albase

        zvec = jnp.zeros((L,), jnp.float32)

        def zbody(r, _):
            for k in range(d // L):
                acc[r, pl.ds(k * L, L)] = zvec
            den[pl.ds(r * L, L)] = zvec
            return 0
        lax.fori_loop(0, ACC_ROWS, zbody, 0)

        pltpu.sync_copy(as_h, as_t)
        pltpu.sync_copy(ad_h.at[pl.ds(albase, ADT)], ad_t)
        pltpu.sync_copy(mx_h, mx_t)
        pltpu.sync_copy(b_h, b_t)

        mrow = mx_t[pl.ds(0, L)]
        msum = mrow[0] + mrow[1]
        mv = jnp.full((L,), msum, jnp.float32)
        mv = jnp.where(mv >= 0.0, mv, mv * NEG_SLOPE)

        def sub_body(sub, _):
            pltpu.sync_copy(esrc_h.at[wid, sub], slist)
            pltpu.sync_copy(edl_h.at[wid, sub], dlist)
            tail = dlist[pl.ds(SLOT - L, L)]
            ng = tail[0]

            def grp_body(g, _):
                cp = pltpu.async_copy(
                    feat_h.at[slist.at[pl.ds(g * G, G)]], rows, sem)
                s16 = slist[pl.ds(g * G, G)]
                dl16 = dlist[pl.ds(g * G, G)]
                av = plsc.load_gather(as_t, [s16])
                dv = plsc.load_gather(ad_t, [dl16 + delta])
                e = av + dv
                e = jnp.where(e >= 0.0, e, e * NEG_SLOPE)
                w = jnp.exp(e - mv)
                cp.wait()
                for j in range(G):
                    dlj = dl16[j]
                    wj = w[j]
                    for k in range(d // L):
                        sl = pl.ds(k * L, L)
                        plsc.addupdate(acc.at[dlj, sl], rows[j, sl] * wj)
                    plsc.addupdate(den.at[pl.ds(dlj * L, L)],
                                   jnp.full((L,), wj, jnp.float32))
                return 0
            lax.fori_loop(0, ng, grp_body, 0)
            return 0
        lax.fori_loop(0, NSUB, sub_body, 0)

        def norm_body(r, _):
            @pl.when(lo + r < hi)
            def _():
                dch = den[pl.ds(r * L, L)]
                dj = jnp.maximum(dch[0], 1e-30)
                for k in range(d // L):
                    sl = pl.ds(k * L, L)
                    val = acc[r, sl] / dj + b_t[sl]
                    if relu:
                        val = jnp.maximum(val, 0.0)
                    rows[0, sl] = val
                pltpu.sync_copy(rows.at[pl.ds(0, 1)],
                                out_h.at[pl.ds(lo + r, 1)])
            return 0
        lax.fori_loop(0, RPT, norm_body, 0)

    return sc_gat


def kernel(x, edge_index, W1, att_src1, att_dst1, b1,
           W2, att_src2, att_dst2, b2):
    n = N_NODES
    e_in = edge_index.shape[1]
    n_edges = e_in + n
    tot = ((n_edges + SUB - 1) // SUB) * SUB

    loop = jnp.arange(n, dtype=jnp.int32)
    src = jnp.concatenate([edge_index[0], loop])
    dst = jnp.concatenate([edge_index[1], loop])
    pad = tot - n_edges
    src_p = jnp.pad(src, (0, pad))
    dst_p = jnp.pad(dst, (0, pad), constant_values=n)

    hid = W1.shape[1]
    out_d = W2.shape[1]
    att1 = jnp.zeros((hid, 128), jnp.float32)
    att1 = att1.at[:, 0].set(att_src1).at[:, 1].set(att_dst1)
    att2 = jnp.zeros((out_d, 128), jnp.float32)
    att2 = att2.at[:, 0].set(att_src2).at[:, 1].set(att_dst2)

    esrc, edl = _make_sc_partition(tot)(src_p, dst_p)

    h1, aux1, mx1 = _tc_matmul(x, W1, att1)
    as1 = jnp.pad(aux1[:, 0], (0, NPAD - n))
    ad1 = jnp.pad(aux1[:, 1], (0, NPAD - n))
    sc1 = _make_sc_gat(hid, tot, relu=True)
    h = sc1(esrc, edl, as1, ad1, mx1.reshape(128), h1, b1)

    h2, aux2, mx2 = _tc_matmul(h, W2, att2)
    as2 = jnp.pad(aux2[:, 0], (0, NPAD - n))
    ad2 = jnp.pad(aux2[:, 1], (0, NPAD - n))
    sc2 = _make_sc_gat(out_d, tot, relu=False)
    x2 = sc2(esrc, edl, as2, ad2, mx2.reshape(128), h2, b2)

    return x2, h
